# baseline (speedup 1.0000x reference)
import jax
import jax.numpy as jnp
from jax import lax
from jax.experimental import pallas as pl
from jax.experimental.pallas import tpu as pltpu


def _build_tc_call(batch, height, width, num_feats, table_rows):
    hw = height * width
    two_d = 2 * num_feats

    def body(row_ref, col_ref, o_ref):
        r_idx = lax.broadcasted_iota(jnp.int32, (hw, width), 0)
        k_idx = lax.broadcasted_iota(jnp.int32, (hw, width), 1)
        s_col = (r_idx % width == k_idx).astype(jnp.float32)
        s_row = (r_idx // width == k_idx).astype(jnp.float32)

        col_hw = col_ref[0:width, :]
        row_hw = row_ref[0:height, :]
        x = lax.dot_general(
            s_col, col_hw, (((1,), (0,)), ((), ())),
            preferred_element_type=jnp.float32,
        )
        y = lax.dot_general(
            s_row, row_hw, (((1,), (0,)), ((), ())),
            preferred_element_type=jnp.float32,
        )
        o_ref[:, 0:num_feats] = x
        o_ref[:, num_feats:two_d] = y

    return pl.pallas_call(
        body,
        grid=(batch,),
        in_specs=[
            pl.BlockSpec((table_rows, num_feats), lambda b: (0, 0)),
            pl.BlockSpec((table_rows, num_feats), lambda b: (0, 0)),
        ],
        out_specs=pl.BlockSpec((hw, two_d), lambda b: (b, 0)),
        out_shape=jax.ShapeDtypeStruct((batch * hw, two_d), jnp.float32),
        compiler_params=pltpu.CompilerParams(
            dimension_semantics=("arbitrary",),
        ),
    )


def kernel(pixel_values, pixel_mask, row_embed, col_embed):
    batch = pixel_values.shape[0]
    height, width = pixel_values.shape[-2:]
    table_rows, num_feats = row_embed.shape
    call = _build_tc_call(batch, height, width, num_feats, table_rows)
    rows = call(row_embed, col_embed)
    out = rows.reshape(batch, height, width, 2 * num_feats)
    return jnp.transpose(out, (0, 3, 1, 2))


# speedup vs baseline: 1.0995x; 1.0995x over previous
import jax
import jax.numpy as jnp
from jax import lax
from jax.experimental import pallas as pl
from jax.experimental.pallas import tpu as pltpu


def _build_tc_call(batch, height, width, num_feats, table_rows):
    hw = height * width
    two_d = 2 * num_feats

    def body(row_ref, col_ref, o_ref):
        r_idx = lax.broadcasted_iota(jnp.int32, (hw, width), 0)
        k_idx = lax.broadcasted_iota(jnp.int32, (hw, width), 1)
        s_col = (r_idx % width == k_idx).astype(jnp.float32)
        s_row = (r_idx // width == k_idx).astype(jnp.float32)

        col_hw = col_ref[0:width, :]
        row_hw = row_ref[0:height, :]
        x = lax.dot_general(
            s_col, col_hw, (((1,), (0,)), ((), ())),
            preferred_element_type=jnp.float32,
        )
        y = lax.dot_general(
            s_row, row_hw, (((1,), (0,)), ((), ())),
            preferred_element_type=jnp.float32,
        )
        for b in range(batch):
            o_ref[b * hw:(b + 1) * hw, 0:num_feats] = x
            o_ref[b * hw:(b + 1) * hw, num_feats:two_d] = y

    return pl.pallas_call(
        body,
        in_specs=[
            pl.BlockSpec((table_rows, num_feats), lambda: (0, 0)),
            pl.BlockSpec((table_rows, num_feats), lambda: (0, 0)),
        ],
        out_specs=pl.BlockSpec((batch * hw, two_d), lambda: (0, 0)),
        out_shape=jax.ShapeDtypeStruct((batch * hw, two_d), jnp.float32),
    )


def kernel(pixel_values, pixel_mask, row_embed, col_embed):
    batch = pixel_values.shape[0]
    height, width = pixel_values.shape[-2:]
    table_rows, num_feats = row_embed.shape
    call = _build_tc_call(batch, height, width, num_feats, table_rows)
    rows = call(row_embed, col_embed)
    out = rows.reshape(batch, height, width, 2 * num_feats)
    return jnp.transpose(out, (0, 3, 1, 2))


# speedup vs baseline: 1.2177x; 1.1075x over previous
import jax
import jax.numpy as jnp
from jax import lax
from jax.experimental import pallas as pl
from jax.experimental.pallas import tpu as pltpu


def _build_tc_call(batch, height, width, num_feats, table_rows):
    hw = height * width
    two_d = 2 * num_feats

    def body(row_ref, col_ref, o_ref, blk, sem):
        r_idx = lax.broadcasted_iota(jnp.int32, (hw, width), 0)
        k_idx = lax.broadcasted_iota(jnp.int32, (hw, width), 1)
        s_col = (r_idx % width == k_idx).astype(jnp.float32)
        s_row = (r_idx // width == k_idx).astype(jnp.float32)

        col_hw = col_ref[0:width, :]
        row_hw = row_ref[0:height, :]
        x = lax.dot_general(
            s_col, col_hw, (((1,), (0,)), ((), ())),
            preferred_element_type=jnp.float32,
        )
        y = lax.dot_general(
            s_row, row_hw, (((1,), (0,)), ((), ())),
            preferred_element_type=jnp.float32,
        )
        blk[:, 0:num_feats] = x
        blk[:, num_feats:two_d] = y
        copies = [
            pltpu.make_async_copy(blk, o_ref.at[pl.ds(b * hw, hw), :], sem)
            for b in range(batch)
        ]
        for cp in copies:
            cp.start()
        for cp in copies:
            cp.wait()

    return pl.pallas_call(
        body,
        in_specs=[
            pl.BlockSpec((table_rows, num_feats), lambda: (0, 0)),
            pl.BlockSpec((table_rows, num_feats), lambda: (0, 0)),
        ],
        out_specs=pl.BlockSpec(memory_space=pltpu.MemorySpace.HBM),
        out_shape=jax.ShapeDtypeStruct((batch * hw, two_d), jnp.float32),
        scratch_shapes=[
            pltpu.VMEM((hw, two_d), jnp.float32),
            pltpu.SemaphoreType.DMA,
        ],
    )


def kernel(pixel_values, pixel_mask, row_embed, col_embed):
    batch = pixel_values.shape[0]
    height, width = pixel_values.shape[-2:]
    table_rows, num_feats = row_embed.shape
    call = _build_tc_call(batch, height, width, num_feats, table_rows)
    rows = call(row_embed, col_embed)
    out = rows.reshape(batch, height, width, 2 * num_feats)
    return jnp.transpose(out, (0, 3, 1, 2))
